# Initial kernel scaffold; baseline (speedup 1.0000x reference)
import functools

import jax
import jax.numpy as jnp
from jax import lax
from jax.experimental import pallas as pl
from jax.experimental.pallas import tpu as pltpu
from jax.experimental.pallas import tpu_sc as plsc

N = 10000
E = 320000
D = 128
H = 32
G = 64
T = 10

EC = 128
NROWS = E // EC
NWORKERS = 32
RPS = N // 16
ZR = 125


def _sc_agg_body(src_hbm, dst_hbm, h_hbm, out_hbm, acc, sbuf, dbuf, rows,
                 zbuf, sem):
    cid = lax.axis_index("c")
    sid = lax.axis_index("s")
    wid = sid * 2 + cid

    zero16 = jnp.zeros((16,), jnp.float32)

    @pl.loop(0, ZR)
    def _zrow(i):
        zbuf[i, pl.ds(0, 16)] = zero16
        zbuf[i, pl.ds(16, 16)] = zero16

    @pl.loop(0, RPS // ZR)
    def _zacc(k):
        pltpu.sync_copy(zbuf, acc.at[pl.ds(sid * RPS + k * ZR, ZR)])

    plsc.subcore_barrier()

    @pl.loop(wid, NROWS, step=NWORKERS)
    def _edge(j):
        pltpu.sync_copy(src_hbm.at[j], sbuf)
        pltpu.sync_copy(dst_hbm.at[j], dbuf)
        pltpu.async_copy(h_hbm.at[sbuf], rows, sem).wait()
        pltpu.sync_copy(rows, acc.at[dbuf], add=True)

    plsc.subcore_barrier()

    @pl.loop(0, RPS // ZR)
    def _out(k):
        pltpu.sync_copy(acc.at[pl.ds(sid * RPS + k * ZR, ZR)], zbuf)
        pltpu.sync_copy(
            zbuf, out_hbm.at[pl.ds(cid * N + sid * RPS + k * ZR, ZR)])


_sc_aggregate = functools.partial(
    pl.kernel,
    out_type=jax.ShapeDtypeStruct((2 * N, H), jnp.float32),
    mesh=plsc.VectorSubcoreMesh(core_axis_name="c", subcore_axis_name="s"),
    scratch_types=[
        pltpu.VMEM_SHARED((N, H), jnp.float32),
        pltpu.VMEM((EC,), jnp.int32),
        pltpu.VMEM((EC,), jnp.int32),
        pltpu.VMEM((EC, H), jnp.float32),
        pltpu.VMEM((ZR, H), jnp.float32),
        pltpu.SemaphoreType.DMA,
    ],
)(_sc_agg_body)


def _bn_relu(h, g, b):
    m = jnp.mean(h, axis=0, keepdims=True)
    v = jnp.mean((h - m) ** 2, axis=0, keepdims=True)
    return jnp.maximum((h - m) / jnp.sqrt(v + 1e-5) * g + b, 0.0)


def _mlp(h, w1, b1, g1, be1, w2, b2, g2, be2):
    h = _bn_relu(
        jnp.dot(h, w1[...], preferred_element_type=jnp.float32) + b1[...],
        g1[...], be1[...])
    h = _bn_relu(
        jnp.dot(h, w2[...], preferred_element_type=jnp.float32) + b2[...],
        g2[...], be2[...])
    return h


def _onehot(b_ref):
    ids = lax.broadcasted_iota(jnp.int32, (N, G), 1)
    return (b_ref[...] == ids).astype(jnp.float32)


def _seg_matmul(oh, z):
    return lax.dot_general(oh, z, (((0,), (0,)), ((), ())),
                           preferred_element_type=jnp.float32)


def _first_body(x_ref, b_ref, w1, b1, g1, be1, w2, b2, g2, be2, lw, lb,
                h_out, o_out):
    h = _mlp(x_ref[...], w1, b1, g1, be1, w2, b2, g2, be2)
    h_out[...] = h
    z = jnp.dot(h, lw[...], preferred_element_type=jnp.float32) + lb[...]
    oh = _onehot(b_ref)
    pooled = _seg_matmul(oh, z)
    cnt = _seg_matmul(oh, jnp.ones((N, T), jnp.float32))
    o_out[...] = pooled / jnp.maximum(cnt, 1.0)


def _conv_body(h_ref, agg_ref, b_ref, w1, b1, g1, be1, w2, b2, g2, be2,
               lw, lb, h_out, o_out):
    a = agg_ref[...]
    hin = h_ref[...] + a[:N] + a[N:]
    h = _mlp(hin, w1, b1, g1, be1, w2, b2, g2, be2)
    h_out[...] = h
    oh = _onehot(b_ref)
    pooled = _seg_matmul(oh, h)
    cnt = _seg_matmul(oh, jnp.ones((N, H), jnp.float32))
    pm = pooled / jnp.maximum(cnt, 1.0)
    o_out[...] = (jnp.dot(pm, lw[...], preferred_element_type=jnp.float32)
                  + lb[...])


def _mlp_args(p):
    r = lambda a: a.reshape(1, -1)
    return (p["w1"], r(p["b1"]), r(p["g1"]), r(p["be1"]),
            p["w2"], r(p["b2"]), r(p["g2"]), r(p["be2"]))


_first_call = pl.pallas_call(
    _first_body,
    out_shape=(
        jax.ShapeDtypeStruct((N, H), jnp.float32),
        jax.ShapeDtypeStruct((G, T), jnp.float32),
    ),
)

_conv_call = pl.pallas_call(
    _conv_body,
    out_shape=(
        jax.ShapeDtypeStruct((N, H), jnp.float32),
        jax.ShapeDtypeStruct((G, T), jnp.float32),
    ),
)


@jax.jit
def kernel(x, edge_index, batch, params):
    src2d = edge_index[0].reshape(NROWS, EC)
    dst2d = edge_index[1].reshape(NROWS, EC)
    b2d = batch.reshape(N, 1)

    h0, out0 = _first_call(x, b2d, *_mlp_args(params["fh"]),
                           params["l0_w"], params["l0_b"].reshape(1, T))
    agg1 = _sc_aggregate(src2d, dst2d, h0)
    h1, out1 = _conv_call(h0, agg1, b2d, *_mlp_args(params["c1"]),
                          params["l1_w"], params["l1_b"].reshape(1, T))
    agg2 = _sc_aggregate(src2d, dst2d, h1)
    _, out2 = _conv_call(h1, agg2, b2d, *_mlp_args(params["c2"]),
                         params["l2_w"], params["l2_b"].reshape(1, T))
    return out0 + out1 + out2


# speedup vs baseline: 7.2347x; 7.2347x over previous
import functools

import jax
import jax.numpy as jnp
from jax import lax
from jax.experimental import pallas as pl
from jax.experimental.pallas import tpu as pltpu
from jax.experimental.pallas import tpu_sc as plsc

N = 10000
E = 320000
D = 128
H = 32
G = 64
T = 10

EC = 128
GB = 8
E_PAD = 327680
NROWS = E_PAD // EC
NGROUPS = NROWS // GB
NWORKERS = 32
N_PAD = 10240
RPS = N_PAD // 16
ZR = 160


def _sc_agg_body(src_hbm, dst_hbm, h_hbm, out_hbm, acc, sbuf, dbuf, rows,
                 zbuf, sem):
    cid = lax.axis_index("c")
    sid = lax.axis_index("s")
    wid = sid * 2 + cid

    zero16 = jnp.zeros((16,), jnp.float32)

    @pl.loop(0, ZR)
    def _zrow(i):
        zbuf[i, pl.ds(0, 16)] = zero16
        zbuf[i, pl.ds(16, 16)] = zero16

    @pl.loop(0, RPS // ZR)
    def _zacc(k):
        pltpu.sync_copy(zbuf, acc.at[pl.ds(sid * RPS + k * ZR, ZR)])

    plsc.subcore_barrier()

    @pl.loop(wid, NGROUPS, step=NWORKERS)
    def _edge(g):
        pltpu.sync_copy(src_hbm.at[pl.ds(g * GB, GB)], sbuf)
        pltpu.sync_copy(dst_hbm.at[pl.ds(g * GB, GB)], dbuf)
        for i in range(GB):
            pltpu.async_copy(h_hbm.at[sbuf.at[i]],
                             rows.at[pl.ds(i * EC, EC)], sem).wait()
            pltpu.sync_copy(rows.at[pl.ds(i * EC, EC)],
                            acc.at[dbuf.at[i]], add=True)

    plsc.subcore_barrier()

    @pl.loop(0, RPS // ZR)
    def _out(k):
        pltpu.sync_copy(acc.at[pl.ds(sid * RPS + k * ZR, ZR)], zbuf)
        pltpu.sync_copy(
            zbuf, out_hbm.at[pl.ds(cid * N_PAD + sid * RPS + k * ZR, ZR)])


@functools.cache
def _sc_aggregate_call():
    return pl.kernel(
        _sc_agg_body,
        out_type=jax.ShapeDtypeStruct((2 * N_PAD, H), jnp.float32),
        mesh=plsc.VectorSubcoreMesh(core_axis_name="c", subcore_axis_name="s"),
        compiler_params=pltpu.CompilerParams(use_tc_tiling_on_sc=False),
        scratch_types=[
            pltpu.VMEM_SHARED((N_PAD, H), jnp.float32),
            pltpu.VMEM((GB, EC), jnp.int32),
            pltpu.VMEM((GB, EC), jnp.int32),
            pltpu.VMEM((GB * EC, H), jnp.float32),
            pltpu.VMEM((ZR, H), jnp.float32),
            pltpu.SemaphoreType.DMA,
        ],
    )


def _bn_relu(h, g, b):
    m = jnp.mean(h, axis=0, keepdims=True)
    v = jnp.mean((h - m) ** 2, axis=0, keepdims=True)
    return jnp.maximum((h - m) / jnp.sqrt(v + 1e-5) * g + b, 0.0)


def _mlp(h, w1, b1, g1, be1, w2, b2, g2, be2):
    h = _bn_relu(
        jnp.dot(h, w1[...], preferred_element_type=jnp.float32) + b1[...],
        g1[...], be1[...])
    h = _bn_relu(
        jnp.dot(h, w2[...], preferred_element_type=jnp.float32) + b2[...],
        g2[...], be2[...])
    return h


def _onehot(b_ref):
    ids = lax.broadcasted_iota(jnp.int32, (N, G), 1)
    return (b_ref[...] == ids).astype(jnp.float32)


def _seg_matmul(oh, z):
    return lax.dot_general(oh, z, (((0,), (0,)), ((), ())),
                           preferred_element_type=jnp.float32)


def _first_body(x_ref, b_ref, w1, b1, g1, be1, w2, b2, g2, be2, lw, lb,
                h_out, o_out):
    h = _mlp(x_ref[...], w1, b1, g1, be1, w2, b2, g2, be2)
    h_out[...] = h
    z = jnp.dot(h, lw[...], preferred_element_type=jnp.float32) + lb[...]
    oh = _onehot(b_ref)
    pooled = _seg_matmul(oh, z)
    cnt = _seg_matmul(oh, jnp.ones((N, T), jnp.float32))
    o_out[...] = pooled / jnp.maximum(cnt, 1.0)


def _conv_body(h_ref, agg_ref, b_ref, w1, b1, g1, be1, w2, b2, g2, be2,
               lw, lb, h_out, o_out):
    a = agg_ref[...]
    hin = h_ref[...] + a[:N] + a[N_PAD:N_PAD + N]
    h = _mlp(hin, w1, b1, g1, be1, w2, b2, g2, be2)
    h_out[...] = h
    oh = _onehot(b_ref)
    pooled = _seg_matmul(oh, h)
    cnt = _seg_matmul(oh, jnp.ones((N, H), jnp.float32))
    pm = pooled / jnp.maximum(cnt, 1.0)
    o_out[...] = (jnp.dot(pm, lw[...], preferred_element_type=jnp.float32)
                  + lb[...])


def _mlp_args(p):
    r = lambda a: a.reshape(1, -1)
    return (p["w1"], r(p["b1"]), r(p["g1"]), r(p["be1"]),
            p["w2"], r(p["b2"]), r(p["g2"]), r(p["be2"]))


_first_call = pl.pallas_call(
    _first_body,
    out_shape=(
        jax.ShapeDtypeStruct((N, H), jnp.float32),
        jax.ShapeDtypeStruct((G, T), jnp.float32),
    ),
)

_conv_call = pl.pallas_call(
    _conv_body,
    out_shape=(
        jax.ShapeDtypeStruct((N, H), jnp.float32),
        jax.ShapeDtypeStruct((G, T), jnp.float32),
    ),
)


@jax.jit
def kernel(x, edge_index, batch, params):
    npad = E_PAD - E
    src_pad = jnp.concatenate(
        [edge_index[0], jnp.zeros((npad,), jnp.int32)])
    dst_pad = jnp.concatenate(
        [edge_index[1], N + (jnp.arange(npad, dtype=jnp.int32) % (N_PAD - N))])
    src2d = src_pad.reshape(NROWS, EC)
    dst2d = dst_pad.reshape(NROWS, EC)
    b2d = batch.reshape(N, 1)

    h0, out0 = _first_call(x, b2d, *_mlp_args(params["fh"]),
                           params["l0_w"], params["l0_b"].reshape(1, T))
    sc_agg = _sc_aggregate_call()
    agg1 = sc_agg(src2d, dst2d, h0)
    h1, out1 = _conv_call(h0, agg1, b2d, *_mlp_args(params["c1"]),
                          params["l1_w"], params["l1_b"].reshape(1, T))
    agg2 = sc_agg(src2d, dst2d, h1)
    _, out2 = _conv_call(h1, agg2, b2d, *_mlp_args(params["c2"]),
                         params["l2_w"], params["l2_b"].reshape(1, T))
    return out0 + out1 + out2
